# baseline (speedup 1.0000x reference)
import jax
import jax.numpy as jnp
from jax import lax
from jax.experimental import pallas as pl
from jax.experimental.pallas import tpu as pltpu
from jax.experimental.pallas import tpu_sc as plsc

_D = 64
_NC = 2
_NS = 16
_NW = _NC * _NS
_CHUNK = 400


def _gather_body(xe_hbm, xo_hbm, table_hbm, out_hbm, idxe_v, idxo_v,
                 rowse0, rowse1, rowso0, rowso1,
                 gsem0, gsem1, osem0, osem1):
    wid = lax.axis_index("s") * _NC + lax.axis_index("c")
    n_half = xe_hbm.shape[0]
    h_per_w = n_half // _NW
    n_chunks = h_per_w // _CHUNK
    n_pairs = n_chunks // 2
    base = wid * h_per_w

    rowse = (rowse0, rowse1)
    rowso = (rowso0, rowso1)
    gsem = (gsem0, gsem1)
    osem = (osem0, osem1)

    pltpu.sync_copy(xe_hbm.at[pl.ds(base, h_per_w)], idxe_v)
    pltpu.sync_copy(xo_hbm.at[pl.ds(base, h_per_w)], idxo_v)

    def gathers_of(g, b):
        sl = pl.ds(g * _CHUNK, _CHUNK)
        return [
            pltpu.make_async_copy(
                table_hbm.at[idxe_v.at[sl]], rowse[b], gsem[b]),
            pltpu.make_async_copy(
                table_hbm.at[idxo_v.at[sl]], rowso[b], gsem[b]),
        ]

    def start_gathers(g, b):
        for c in gathers_of(g, b):
            c.start()

    def wait_gathers(g, b):
        for c in gathers_of(g, b):
            c.wait()

    def stores_of(g, b):
        sl = pl.ds(base + g * _CHUNK, _CHUNK)
        return [
            pltpu.make_async_copy(
                rowse[b], out_hbm.at[sl, pl.ds(0, _D)], osem[b]),
            pltpu.make_async_copy(
                rowso[b], out_hbm.at[sl, pl.ds(_D, _D)], osem[b]),
        ]

    def store_of(g, b):
        class _Pair:
            def __init__(self, cs):
                self.cs = cs
            def start(self):
                for c in self.cs:
                    c.start()
            def wait(self):
                for c in self.cs:
                    c.wait()
        return _Pair(stores_of(g, b))

    start_gathers(0, 0)

    def pair(p, carry):
        g0 = 2 * p
        wait_gathers(g0, 0)
        @pl.when(p > 0)
        def _():
            store_of(g0 - 1, 1).wait()
        start_gathers(g0 + 1, 1)
        store_of(g0, 0).start()
        wait_gathers(g0 + 1, 1)
        @pl.when(p < n_pairs - 1)
        def _():
            store_of(g0, 0).wait()
            start_gathers(g0 + 2, 0)
        store_of(g0 + 1, 1).start()
        return carry

    lax.fori_loop(0, n_pairs, pair, 0)
    store_of(n_chunks - 2, 0).wait()
    store_of(n_chunks - 1, 1).wait()


def kernel(x, cache_geg):
    b, s = x.shape
    n_total = b * s
    flat2 = x.reshape(-1, 2)
    xe = flat2[:, 0]
    xo = flat2[:, 1]
    h_per_w = (n_total // 2) // _NW
    mesh = plsc.VectorSubcoreMesh(core_axis_name="c", subcore_axis_name="s")
    gather = pl.kernel(
        _gather_body,
        mesh=mesh,
        compiler_params=pltpu.CompilerParams(use_tc_tiling_on_sc=False),
        out_type=jax.ShapeDtypeStruct((n_total // 2, 2 * _D), jnp.float32),
        scratch_types=[
            pltpu.VMEM((h_per_w,), jnp.int32),
            pltpu.VMEM((h_per_w,), jnp.int32),
            pltpu.VMEM((_CHUNK, _D), jnp.float32),
            pltpu.VMEM((_CHUNK, _D), jnp.float32),
            pltpu.VMEM((_CHUNK, _D), jnp.float32),
            pltpu.VMEM((_CHUNK, _D), jnp.float32),
            pltpu.SemaphoreType.DMA,
            pltpu.SemaphoreType.DMA,
            pltpu.SemaphoreType.DMA,
            pltpu.SemaphoreType.DMA,
        ],
    )
    out = gather(xe, xo, cache_geg)
    return out.reshape(b, s, _D)


# speedup vs baseline: 1.0173x; 1.0173x over previous
import jax
import jax.numpy as jnp
from jax import lax
from jax.experimental import pallas as pl
from jax.experimental.pallas import tpu as pltpu
from jax.experimental.pallas import tpu_sc as plsc

_D = 64
_DP = 128
_NC = 2
_NS = 16
_NW = _NC * _NS
_CHUNK = 128


def _gather_body(xe_hbm, xo_hbm, table_hbm, out_hbm, idxe_v, idxo_v,
                 rowse0, rowse1, rowso0, rowso1,
                 gsem0, gsem1, osem0, osem1):
    wid = lax.axis_index("s") * _NC + lax.axis_index("c")
    n_half = xe_hbm.shape[0]
    h_per_w = n_half // _NW
    n_chunks = h_per_w // _CHUNK
    n_pairs = n_chunks // 2
    base = wid * h_per_w

    rowse = (rowse0, rowse1)
    rowso = (rowso0, rowso1)
    gsem = (gsem0, gsem1)
    osem = (osem0, osem1)

    pltpu.sync_copy(xe_hbm.at[pl.ds(base, h_per_w)], idxe_v)
    pltpu.sync_copy(xo_hbm.at[pl.ds(base, h_per_w)], idxo_v)

    def gathers_of(g, b):
        sl = pl.ds(g * _CHUNK, _CHUNK)
        return [
            pltpu.make_async_copy(
                table_hbm.at[idxe_v.at[sl]], rowse[b], gsem[b]),
            pltpu.make_async_copy(
                table_hbm.at[idxo_v.at[sl]], rowso[b], gsem[b]),
        ]

    def start_gathers(g, b):
        for c in gathers_of(g, b):
            c.start()

    def wait_gathers(g, b):
        for c in gathers_of(g, b):
            c.wait()

    def pack(b):
        def row(i, c):
            for j in range(_D // 16):
                rowse[b][i, pl.ds(_D + j * 16, 16)] = (
                    rowso[b][i, pl.ds(j * 16, 16)])
            return c
        lax.fori_loop(0, _CHUNK, row, 0)

    def store_of(g, b):
        return pltpu.make_async_copy(
            rowse[b], out_hbm.at[pl.ds(base + g * _CHUNK, _CHUNK)], osem[b])

    start_gathers(0, 0)

    def pair(p, carry):
        g0 = 2 * p
        wait_gathers(g0, 0)
        @pl.when(p > 0)
        def _():
            store_of(g0 - 1, 1).wait()
        start_gathers(g0 + 1, 1)
        pack(0)
        store_of(g0, 0).start()
        wait_gathers(g0 + 1, 1)
        @pl.when(p < n_pairs - 1)
        def _():
            store_of(g0, 0).wait()
            start_gathers(g0 + 2, 0)
        pack(1)
        store_of(g0 + 1, 1).start()
        return carry

    lax.fori_loop(0, n_pairs, pair, 0)
    store_of(n_chunks - 2, 0).wait()
    store_of(n_chunks - 1, 1).wait()


def kernel(x, cache_geg):
    b, s = x.shape
    n_total = b * s
    flat = x.reshape(-1)
    xe = flat[0::2]
    xo = flat[1::2]
    table128 = jnp.pad(cache_geg, ((0, 0), (0, _DP - _D)))
    h_per_w = (n_total // 2) // _NW
    mesh = plsc.VectorSubcoreMesh(core_axis_name="c", subcore_axis_name="s")
    gather = pl.kernel(
        _gather_body,
        mesh=mesh,
        out_type=jax.ShapeDtypeStruct((n_total // 2, _DP), jnp.float32),
        scratch_types=[
            pltpu.VMEM((h_per_w,), jnp.int32),
            pltpu.VMEM((h_per_w,), jnp.int32),
            pltpu.VMEM((_CHUNK, _DP), jnp.float32),
            pltpu.VMEM((_CHUNK, _DP), jnp.float32),
            pltpu.VMEM((_CHUNK, _DP), jnp.float32),
            pltpu.VMEM((_CHUNK, _DP), jnp.float32),
            pltpu.SemaphoreType.DMA,
            pltpu.SemaphoreType.DMA,
            pltpu.SemaphoreType.DMA,
            pltpu.SemaphoreType.DMA,
        ],
    )
    out = gather(xe, xo, table128)
    return out.reshape(b, s, _D)


# speedup vs baseline: 1.2804x; 1.2585x over previous
import jax
import jax.numpy as jnp
from jax import lax
from jax.experimental import pallas as pl
from jax.experimental.pallas import tpu as pltpu
from jax.experimental.pallas import tpu_sc as plsc

_D = 64
_NC = 2
_NS = 16
_NW = _NC * _NS
_S = 200


def _gather_body(x_hbm, table_hbm, out_hbm, idx_v, rows0, rows1,
                 gsem0, gsem1, osem0, osem1):
    wid = lax.axis_index("s") * _NC + lax.axis_index("c")
    n_batch = x_hbm.shape[0]
    b_per_w = n_batch // _NW
    n_pairs = b_per_w // 2
    base = wid * b_per_w

    rows = (rows0, rows1)
    gsem = (gsem0, gsem1)
    osem = (osem0, osem1)

    pltpu.sync_copy(x_hbm.at[pl.ds(base, b_per_w)], idx_v)

    def gather_of(g, b):
        return pltpu.make_async_copy(
            table_hbm.at[idx_v.at[g]], rows[b], gsem[b])

    def store_of(g, b):
        return pltpu.make_async_copy(
            rows[b], out_hbm.at[base + g], osem[b])

    gather_of(0, 0).start()

    def pair(p, carry):
        g0 = 2 * p
        gather_of(g0, 0).wait()
        @pl.when(p > 0)
        def _():
            store_of(g0 - 1, 1).wait()
        gather_of(g0 + 1, 1).start()
        store_of(g0, 0).start()
        gather_of(g0 + 1, 1).wait()
        @pl.when(p < n_pairs - 1)
        def _():
            store_of(g0, 0).wait()
            gather_of(g0 + 2, 0).start()
        store_of(g0 + 1, 1).start()
        return carry

    lax.fori_loop(0, n_pairs, pair, 0)
    store_of(b_per_w - 2, 0).wait()
    store_of(b_per_w - 1, 1).wait()


def kernel(x, cache_geg):
    b, s = x.shape
    b_per_w = b // _NW
    mesh = plsc.VectorSubcoreMesh(core_axis_name="c", subcore_axis_name="s")
    gather = pl.kernel(
        _gather_body,
        mesh=mesh,
        compiler_params=pltpu.CompilerParams(use_tc_tiling_on_sc=False),
        out_type=jax.ShapeDtypeStruct((b, s, _D), jnp.float32),
        scratch_types=[
            pltpu.VMEM((b_per_w, _S), jnp.int32),
            pltpu.VMEM((_S, _D), jnp.float32),
            pltpu.VMEM((_S, _D), jnp.float32),
            pltpu.SemaphoreType.DMA,
            pltpu.SemaphoreType.DMA,
            pltpu.SemaphoreType.DMA,
            pltpu.SemaphoreType.DMA,
        ],
    )
    return gather(x, cache_geg)


# speedup vs baseline: 1.3585x; 1.0610x over previous
import jax
import jax.numpy as jnp
from jax import lax
from jax.experimental import pallas as pl
from jax.experimental.pallas import tpu as pltpu
from jax.experimental.pallas import tpu_sc as plsc

_D = 64
_NC = 2
_NS = 16
_NW = _NC * _NS
_CHUNK = 800
_S = 200
_BPC = _CHUNK // _S


def _gather_body(x_hbm, table_hbm, out_hbm, idx_v, rows0, rows1,
                 gsem0, gsem1, osem0, osem1):
    wid = lax.axis_index("s") * _NC + lax.axis_index("c")
    n_total = x_hbm.shape[0]
    b_per_w = n_total // _NW
    n_chunks = b_per_w // _CHUNK
    n_pairs = n_chunks // 2
    base = wid * b_per_w

    rows = (rows0, rows1)
    gsem = (gsem0, gsem1)
    osem = (osem0, osem1)

    pltpu.sync_copy(x_hbm.at[pl.ds(base, b_per_w)], idx_v)

    def gather_of(g, b):
        return pltpu.make_async_copy(
            table_hbm.at[idx_v.at[pl.ds(g * _CHUNK, _CHUNK)]],
            rows[b], gsem[b])

    def stores_of(g, b):
        batch0 = (base + g * _CHUNK) // _S
        return [
            pltpu.make_async_copy(
                rows[b].at[pl.ds(j * _S, _S)], out_hbm.at[batch0 + j], osem[b])
            for j in range(_BPC)
        ]

    def start_stores(g, b):
        for c in stores_of(g, b):
            c.start()

    def wait_stores(g, b):
        for c in stores_of(g, b):
            c.wait()

    gather_of(0, 0).start()

    def pair(p, carry):
        g0 = 2 * p
        gather_of(g0, 0).wait()
        @pl.when(p > 0)
        def _():
            wait_stores(g0 - 1, 1)
        gather_of(g0 + 1, 1).start()
        start_stores(g0, 0)
        gather_of(g0 + 1, 1).wait()
        @pl.when(p < n_pairs - 1)
        def _():
            wait_stores(g0, 0)
            gather_of(g0 + 2, 0).start()
        start_stores(g0 + 1, 1)
        return carry

    lax.fori_loop(0, n_pairs, pair, 0)
    wait_stores(n_chunks - 2, 0)
    wait_stores(n_chunks - 1, 1)


def kernel(x, cache_geg):
    b, s = x.shape
    flat = x.reshape(-1)
    n_total = b * s
    b_per_w = n_total // _NW
    mesh = plsc.VectorSubcoreMesh(core_axis_name="c", subcore_axis_name="s")
    gather = pl.kernel(
        _gather_body,
        mesh=mesh,
        compiler_params=pltpu.CompilerParams(use_tc_tiling_on_sc=False),
        out_type=jax.ShapeDtypeStruct((b, s, _D), jnp.float32),
        scratch_types=[
            pltpu.VMEM((b_per_w,), jnp.int32),
            pltpu.VMEM((_CHUNK, _D), jnp.float32),
            pltpu.VMEM((_CHUNK, _D), jnp.float32),
            pltpu.SemaphoreType.DMA,
            pltpu.SemaphoreType.DMA,
            pltpu.SemaphoreType.DMA,
            pltpu.SemaphoreType.DMA,
        ],
    )
    return gather(flat, cache_geg)
